# baseline (speedup 1.0000x reference)
import functools

import jax
import jax.numpy as jnp
from jax import lax
from jax.experimental import pallas as pl
from jax.experimental.pallas import tpu as pltpu
from jax.experimental.pallas import tpu_sc as plsc

N = 10000
E = 160000
NBT = 4
F0 = 256
H = 512
FC = 16
ACC_ROWS = NBT * N + 64
ROWS_PER_TILE = ACC_ROWS // 16
EPAD = 163840
KB = 2048
NBATCH = (EPAD // 16) // KB


OUT_ROWS = NBT * N
DUMP_PER_TILE = OUT_ROWS // 16


def _make_sc_scatter(nc):
    cpc = nc // 2
    mesh = plsc.VectorSubcoreMesh(core_axis_name="c", subcore_axis_name="s")

    @functools.partial(
        pl.kernel,
        mesh=mesh,
        compiler_params=pltpu.CompilerParams(use_tc_tiling_on_sc=False),
        out_type=jax.ShapeDtypeStruct((nc, OUT_ROWS, FC), jnp.float32),
        scratch_types=[
            pltpu.VMEM((NBATCH, KB), jnp.int32),
            pltpu.VMEM((NBATCH, KB), jnp.int32),
            [pltpu.VMEM((KB, FC), jnp.float32)] * 2,
            pltpu.VMEM_SHARED((ACC_ROWS, FC), jnp.float32),
            [pltpu.SemaphoreType.DMA] * 2,
        ],
    )
    def sc_fn(table_hbm, beg_hbm, dst_hbm, zeros_hbm, agg_hbm,
              beg_v, dst_v, rows_v, acc_sh, sems):
        cid = lax.axis_index("c")
        sid = lax.axis_index("s")
        pltpu.sync_copy(beg_hbm.at[pl.ds(sid * NBATCH, NBATCH)], beg_v)
        pltpu.sync_copy(dst_hbm.at[pl.ds(sid * NBATCH, NBATCH)], dst_v)

        for k in range(cpc):
            chunk = cid * cpc + k
            tview = table_hbm.at[pl.ds(chunk * N, N)]
            pltpu.sync_copy(
                zeros_hbm.at[pl.ds(sid * DUMP_PER_TILE, DUMP_PER_TILE)],
                acc_sh.at[pl.ds(sid * DUMP_PER_TILE, DUMP_PER_TILE)])
            plsc.subcore_barrier()
            cp = pltpu.async_copy(tview.at[beg_v.at[0]], rows_v[0], sems[0])
            for b in range(NBATCH):
                nxt = None
                if b + 1 < NBATCH:
                    nxt = pltpu.async_copy(tview.at[beg_v.at[b + 1]],
                                           rows_v[(b + 1) % 2],
                                           sems[(b + 1) % 2])
                cp.wait()
                pltpu.sync_copy(rows_v[b % 2], acc_sh.at[dst_v.at[b]],
                                add=True)
                cp = nxt
            plsc.subcore_barrier()
            pltpu.sync_copy(
                acc_sh.at[pl.ds(sid * DUMP_PER_TILE, DUMP_PER_TILE)],
                agg_hbm.at[chunk, pl.ds(sid * DUMP_PER_TILE, DUMP_PER_TILE)])
            plsc.subcore_barrier()

    return sc_fn


_sc_scatter_l0 = _make_sc_scatter(F0 // FC)
_sc_scatter_l1 = _make_sc_scatter(H // FC)


def _relayout_body(x_ref, o_ref):
    for c in range(F0 // FC):
        o_ref[c] = x_ref[:, c * FC:(c + 1) * FC]


def _flat_idx_body(end_ref, bt_ref, o_ref):
    e = end_ref[...]
    hi = (e >= N // 2).astype(jnp.int32)
    o_ref[...] = (e - hi * (N // 2)) * (2 * NBT) + hi * NBT + (bt_ref[...] % NBT)


def _mm_self_body(x_ref, w_ref, b_ref, o_ref):
    o_ref[...] = jnp.dot(x_ref[...], w_ref[...],
                         preferred_element_type=jnp.float32) + b_ref[...]


def _mm1_body(s_ref, agg_ref, wa_ref, eh_ref, ehcm_ref):
    for h2 in range(2):
        acc = s_ref[h2]
        for c in range(F0 // FC):
            blk = agg_ref[c][:, h2 * NBT * FC:(h2 + 1) * NBT * FC]
            acc += jnp.dot(blk, wa_ref[c], preferred_element_type=jnp.float32)
        eh = jnp.where(acc > 0, acc, jnp.exp(acc) - 1.0)
        eh_ref[h2] = eh
        for c in range(H // FC):
            ehcm_ref[c, h2] = eh[:, c * FC:(c + 1) * FC]


def _mm2_body(s_ref, agg_ref, wa_ref, o_ref):
    for h2 in range(2):
        acc = s_ref[h2]
        for c in range(H // FC):
            blk = agg_ref[c][:, h2 * NBT * FC:(h2 + 1) * NBT * FC]
            acc += jnp.dot(blk, wa_ref[c], preferred_element_type=jnp.float32)
        o_ref[h2] = acc


def kernel(atom_features, W0, b0, W1, b1, bond_info):
    x = atom_features
    begin = bond_info[:, 0]
    end = bond_info[:, 1]
    btype = bond_info[:, 2]

    bn = 1000
    x_cm = pl.pallas_call(
        _relayout_body,
        grid=(N // bn,),
        in_specs=[pl.BlockSpec((bn, F0), lambda i: (i, 0))],
        out_specs=pl.BlockSpec((F0 // FC, bn, FC), lambda i: (0, i, 0)),
        out_shape=jax.ShapeDtypeStruct((F0 // FC, N, FC), jnp.float32),
    )(x).reshape((F0 // FC) * N, FC)

    end2d = end.reshape(E // 128, 128)
    bt2d = btype.reshape(E // 128, 128)
    flat2d = pl.pallas_call(
        _flat_idx_body,
        out_shape=jax.ShapeDtypeStruct((E // 128, 128), jnp.int32),
    )(end2d, bt2d)

    npad = EPAD - E
    pad_ar = jnp.arange(npad, dtype=jnp.int32)
    beg_pad = jnp.concatenate([begin, pad_ar % 256]).reshape(EPAD // KB, KB)
    dst_pad = jnp.concatenate(
        [flat2d.reshape(E), NBT * N + pad_ar % 64]).reshape(EPAD // KB, KB)

    W0s = W0[:F0]
    W1s = W1[:H]

    zeros_acc = jnp.zeros((OUT_ROWS, FC), jnp.float32)
    agg0 = _sc_scatter_l0(x_cm, beg_pad, dst_pad, zeros_acc)
    bns = 1000
    s0 = pl.pallas_call(
        _mm_self_body,
        grid=(N // bns,),
        in_specs=[
            pl.BlockSpec((bns, F0), lambda i: (i, 0)),
            pl.BlockSpec((F0, H), lambda i: (0, 0)),
            pl.BlockSpec((1, H), lambda i: (0, 0)),
        ],
        out_specs=pl.BlockSpec((bns, H), lambda i: (i, 0)),
        out_shape=jax.ShapeDtypeStruct((N, H), jnp.float32),
    )(x, W0s, b0.reshape(1, H))
    HN = N // 2
    agg0p = agg0.reshape(F0 // FC, HN, 8 * FC)

    W0a = W0[F0:].reshape(NBT, F0 // FC, FC, H).transpose(1, 0, 2, 3)
    W0a = W0a.reshape(F0 // FC, NBT * FC, H)
    W1a = W1[H:].reshape(NBT, H // FC, FC, H).transpose(1, 0, 2, 3)
    W1a = W1a.reshape(H // FC, NBT * FC, H)

    bn1 = 200
    s02 = s0.reshape(2, HN, H)
    eh, ehcm = pl.pallas_call(
        _mm1_body,
        grid=(HN // bn1,),
        in_specs=[
            pl.BlockSpec((2, bn1, H), lambda i: (0, i, 0)),
            pl.BlockSpec((F0 // FC, bn1, 8 * FC), lambda i: (0, i, 0)),
            pl.BlockSpec((F0 // FC, NBT * FC, H), lambda i: (0, 0, 0)),
        ],
        out_specs=[
            pl.BlockSpec((2, bn1, H), lambda i: (0, i, 0)),
            pl.BlockSpec((H // FC, 2, bn1, FC), lambda i: (0, 0, i, 0)),
        ],
        out_shape=[
            jax.ShapeDtypeStruct((2, HN, H), jnp.float32),
            jax.ShapeDtypeStruct((H // FC, 2, HN, FC), jnp.float32),
        ],
    )(s02, agg0p, W0a)

    agg1 = _sc_scatter_l1(ehcm.reshape((H // FC) * N, FC), beg_pad, dst_pad,
                          zeros_acc)
    s1 = pl.pallas_call(
        _mm_self_body,
        grid=(N // bns,),
        in_specs=[
            pl.BlockSpec((bns, H), lambda i: (i, 0)),
            pl.BlockSpec((H, H), lambda i: (0, 0)),
            pl.BlockSpec((1, H), lambda i: (0, 0)),
        ],
        out_specs=pl.BlockSpec((bns, H), lambda i: (i, 0)),
        out_shape=jax.ShapeDtypeStruct((N, H), jnp.float32),
    )(eh.reshape(N, H), W1s, b1.reshape(1, H))
    agg1p = agg1.reshape(H // FC, HN, 8 * FC)

    bn2 = 1000
    out = pl.pallas_call(
        _mm2_body,
        grid=(HN // bn2,),
        in_specs=[
            pl.BlockSpec((2, bn2, H), lambda i: (0, i, 0)),
            pl.BlockSpec((H // FC, bn2, 8 * FC), lambda i: (0, i, 0)),
            pl.BlockSpec((H // FC, NBT * FC, H), lambda i: (0, 0, 0)),
        ],
        out_specs=pl.BlockSpec((2, bn2, H), lambda i: (0, i, 0)),
        out_shape=jax.ShapeDtypeStruct((2, HN, H), jnp.float32),
    )(s1.reshape(2, HN, H), agg1p, W1a)
    return out.reshape(N, H)


# speedup vs baseline: 1.0856x; 1.0856x over previous
import functools

import jax
import jax.numpy as jnp
from jax import lax
from jax.experimental import pallas as pl
from jax.experimental.pallas import tpu as pltpu
from jax.experimental.pallas import tpu_sc as plsc

N = 10000
E = 160000
NBT = 4
F0 = 256
H = 512
FC = 16
ACC_ROWS = NBT * N + 64
ROWS_PER_TILE = ACC_ROWS // 16
EPAD = 163840
KB = 2048
NBATCH = (EPAD // 16) // KB


OUT_ROWS = NBT * N
DUMP_PER_TILE = OUT_ROWS // 16


def _make_sc_scatter(nc):
    cpc = nc // 2
    mesh = plsc.VectorSubcoreMesh(core_axis_name="c", subcore_axis_name="s")

    @functools.partial(
        pl.kernel,
        mesh=mesh,
        compiler_params=pltpu.CompilerParams(use_tc_tiling_on_sc=False),
        out_type=jax.ShapeDtypeStruct((nc, OUT_ROWS, FC), jnp.float32),
        scratch_types=[
            pltpu.VMEM((NBATCH, KB), jnp.int32),
            pltpu.VMEM((NBATCH, KB), jnp.int32),
            [pltpu.VMEM((KB, FC), jnp.float32)] * 2,
            pltpu.VMEM_SHARED((ACC_ROWS, FC), jnp.float32),
            [pltpu.SemaphoreType.DMA] * 2,
        ],
    )
    def sc_fn(table_hbm, beg_hbm, dst_hbm, zeros_hbm, agg_hbm,
              beg_v, dst_v, rows_v, acc_sh, sems):
        cid = lax.axis_index("c")
        sid = lax.axis_index("s")
        my_rows = pl.ds(sid * DUMP_PER_TILE, DUMP_PER_TILE)
        pltpu.sync_copy(beg_hbm.at[pl.ds(sid * NBATCH, NBATCH)], beg_v)
        pltpu.sync_copy(dst_hbm.at[pl.ds(sid * NBATCH, NBATCH)], dst_v)

        def tview(k):
            return table_hbm.at[pl.ds((cid * cpc + k) * N, N)]

        def gather(t):
            k, b = divmod(t, NBATCH)
            return pltpu.async_copy(tview(k).at[beg_v.at[b]],
                                    rows_v[t % 2], sems[t % 2])

        pltpu.sync_copy(zeros_hbm.at[my_rows], acc_sh.at[my_rows])
        cp = gather(0)
        plsc.subcore_barrier()
        total = cpc * NBATCH
        for t in range(total):
            k, b = divmod(t, NBATCH)
            nxt = gather(t + 1) if t + 1 < total else None
            cp.wait()
            pltpu.sync_copy(rows_v[t % 2], acc_sh.at[dst_v.at[b]], add=True)
            cp = nxt
            if b == NBATCH - 1:
                plsc.subcore_barrier()
                pltpu.sync_copy(
                    acc_sh.at[my_rows],
                    agg_hbm.at[cid * cpc + k, my_rows])
                if k + 1 < cpc:
                    pltpu.sync_copy(zeros_hbm.at[my_rows], acc_sh.at[my_rows])
                    plsc.subcore_barrier()

    return sc_fn


_sc_scatter_l0 = _make_sc_scatter(F0 // FC)
_sc_scatter_l1 = _make_sc_scatter(H // FC)


def _relayout_body(x_ref, o_ref):
    for c in range(F0 // FC):
        o_ref[c] = x_ref[:, c * FC:(c + 1) * FC]


def _flat_idx_body(end_ref, bt_ref, o_ref):
    e = end_ref[...]
    hi = (e >= N // 2).astype(jnp.int32)
    o_ref[...] = (e - hi * (N // 2)) * (2 * NBT) + hi * NBT + (bt_ref[...] % NBT)


def _mm1_body(x_ref, agg_ref, ws_ref, wa_ref, b_ref, eh_ref, ehcm_ref):
    for h2 in range(2):
        acc = jnp.dot(x_ref[h2], ws_ref[...],
                      preferred_element_type=jnp.float32)
        for c in range(F0 // FC):
            blk = agg_ref[c][:, h2 * NBT * FC:(h2 + 1) * NBT * FC]
            acc += jnp.dot(blk, wa_ref[c], preferred_element_type=jnp.float32)
        h = acc + b_ref[...]
        eh = jnp.where(h > 0, h, jnp.exp(h) - 1.0)
        eh_ref[h2] = eh
        for c in range(H // FC):
            ehcm_ref[c, h2] = eh[:, c * FC:(c + 1) * FC]


def _mm2_body(x_ref, agg_ref, ws_ref, wa_ref, b_ref, o_ref):
    for h2 in range(2):
        acc = jnp.dot(x_ref[h2], ws_ref[...],
                      preferred_element_type=jnp.float32)
        for c in range(H // FC):
            blk = agg_ref[c][:, h2 * NBT * FC:(h2 + 1) * NBT * FC]
            acc += jnp.dot(blk, wa_ref[c], preferred_element_type=jnp.float32)
        o_ref[h2] = acc + b_ref[...]


def kernel(atom_features, W0, b0, W1, b1, bond_info):
    x = atom_features
    begin = bond_info[:, 0]
    end = bond_info[:, 1]
    btype = bond_info[:, 2]

    bn = 1000
    x_cm = pl.pallas_call(
        _relayout_body,
        grid=(N // bn,),
        in_specs=[pl.BlockSpec((bn, F0), lambda i: (i, 0))],
        out_specs=pl.BlockSpec((F0 // FC, bn, FC), lambda i: (0, i, 0)),
        out_shape=jax.ShapeDtypeStruct((F0 // FC, N, FC), jnp.float32),
    )(x).reshape((F0 // FC) * N, FC)

    end2d = end.reshape(E // 128, 128)
    bt2d = btype.reshape(E // 128, 128)
    flat2d = pl.pallas_call(
        _flat_idx_body,
        out_shape=jax.ShapeDtypeStruct((E // 128, 128), jnp.int32),
    )(end2d, bt2d)

    npad = EPAD - E
    pad_ar = jnp.arange(npad, dtype=jnp.int32)
    beg_pad = jnp.concatenate([begin, pad_ar % 256]).reshape(EPAD // KB, KB)
    dst_pad = jnp.concatenate(
        [flat2d.reshape(E), NBT * N + pad_ar % 64]).reshape(EPAD // KB, KB)

    zeros_acc = jnp.zeros((OUT_ROWS, FC), jnp.float32)
    agg0 = _sc_scatter_l0(x_cm, beg_pad, dst_pad, zeros_acc)
    HN = N // 2
    agg0p = agg0.reshape(F0 // FC, HN, 8 * FC)

    W0s = W0[:F0]
    W0a = W0[F0:].reshape(NBT, F0 // FC, FC, H).transpose(1, 0, 2, 3)
    W0a = W0a.reshape(F0 // FC, NBT * FC, H)
    W1s = W1[:H]
    W1a = W1[H:].reshape(NBT, H // FC, FC, H).transpose(1, 0, 2, 3)
    W1a = W1a.reshape(H // FC, NBT * FC, H)

    bn1 = 200
    x2 = x.reshape(2, HN, F0)
    eh, ehcm = pl.pallas_call(
        _mm1_body,
        grid=(HN // bn1,),
        in_specs=[
            pl.BlockSpec((2, bn1, F0), lambda i: (0, i, 0)),
            pl.BlockSpec((F0 // FC, bn1, 8 * FC), lambda i: (0, i, 0)),
            pl.BlockSpec((F0, H), lambda i: (0, 0)),
            pl.BlockSpec((F0 // FC, NBT * FC, H), lambda i: (0, 0, 0)),
            pl.BlockSpec((1, H), lambda i: (0, 0)),
        ],
        out_specs=[
            pl.BlockSpec((2, bn1, H), lambda i: (0, i, 0)),
            pl.BlockSpec((H // FC, 2, bn1, FC), lambda i: (0, 0, i, 0)),
        ],
        out_shape=[
            jax.ShapeDtypeStruct((2, HN, H), jnp.float32),
            jax.ShapeDtypeStruct((H // FC, 2, HN, FC), jnp.float32),
        ],
    )(x2, agg0p, W0s, W0a, b0.reshape(1, H))

    agg1 = _sc_scatter_l1(ehcm.reshape((H // FC) * N, FC), beg_pad, dst_pad,
                          zeros_acc)
    agg1p = agg1.reshape(H // FC, HN, 8 * FC)

    bn2 = 1000
    out = pl.pallas_call(
        _mm2_body,
        grid=(HN // bn2,),
        in_specs=[
            pl.BlockSpec((2, bn2, H), lambda i: (0, i, 0)),
            pl.BlockSpec((H // FC, bn2, 8 * FC), lambda i: (0, i, 0)),
            pl.BlockSpec((H, H), lambda i: (0, 0)),
            pl.BlockSpec((H // FC, NBT * FC, H), lambda i: (0, 0, 0)),
            pl.BlockSpec((1, H), lambda i: (0, 0)),
        ],
        out_specs=pl.BlockSpec((2, bn2, H), lambda i: (0, i, 0)),
        out_shape=jax.ShapeDtypeStruct((2, HN, H), jnp.float32),
    )(eh, agg1p, W1s, W1a, b1.reshape(1, H))
    return out.reshape(N, H)
